# baseline (speedup 1.0000x reference)
import functools

import jax
import jax.numpy as jnp
import numpy as np
from jax import lax
from jax.experimental import pallas as pl
from jax.experimental.pallas import tpu as pltpu
from jax.experimental.pallas import tpu_sc as plsc

N = 100000
E = 1600000
H = 32
G = 64
NC = 2
NS = 16
NW = NC * NS
EPT = E // NW
VPT = EPT // 16
BMULT = 41
BSHIFT = 17
MAXROWS = 3197
ACCROWS = 3328
DUMROW = ACCROWS - 1
NPAD = 128
SCHUNK = 256
MCHUNK = 256
NBLK = 4000

_BSTARTS_NP = np.minimum(
    (np.arange(48, dtype=np.int64) * (1 << BSHIFT) + BMULT - 1) // BMULT, N
).astype(np.int32)


def _mesh():
    return plsc.VectorSubcoreMesh(core_axis_name="c", subcore_axis_name="s")


def _wid():
    return lax.axis_index("s") * NC + lax.axis_index("c")


def _take(x, i):
    return x.at[i].get(mode="promise_in_bounds")


def _iota():
    return lax.iota(jnp.int32, 16)


def _bucket(d):
    return (d * BMULT) >> BSHIFT


def _m8(x):
    return pl.multiple_of(x, 8)


def _vsum(v):
    iota = _iota()
    for d in (1, 2, 4, 8):
        v = v + _take(v, jnp.bitwise_xor(iota, d))
    return v


def _ssum(v):
    return _vsum(v)[0]


def _prefix_counts(cbuf, w):
    iota = _iota()
    zi = jnp.zeros((16,), jnp.int32)

    def pf(p, carry):
        s0, s1 = carry
        v = cbuf[pl.ds(_m8(p * 16), 16)]
        b = jnp.bitwise_and(p * 16 + iota, 31)
        s0 = s0 + jnp.where(b < w, v, zi)
        s1 = s1 + jnp.where(b == w, v, zi)
        return (s0, s1)

    s0, s1 = lax.fori_loop(0, NW * 32 // 16, pf, (zi, zi))
    return _ssum(s0), _ssum(s1)


def _hist_body(dst_hbm, counts_hbm, cc_hbm, dbuf, cnt, cntc):
    w = _wid()
    pltpu.sync_copy(dst_hbm.at[pl.ds(_m8(w * EPT), EPT)], dbuf)
    iota = _iota()
    z = jnp.zeros((16,), jnp.int32)
    ones = jnp.ones((16,), jnp.int32)
    for p in range(32):
        cnt[p, pl.ds(0, 16)] = z
    onehots = [jnp.where(iota == k, 1, 0) for k in range(16)]

    def chunk(c, _):
        def body(v, _):
            d = dbuf[pl.ds(_m8(c * 80 + v * 16), 16)]
            b = _bucket(d)
            for k in range(16):
                bk = b[k]
                cnt[bk, pl.ds(0, 16)] = cnt[bk, pl.ds(0, 16)] + onehots[k]
            return 0

        lax.fori_loop(0, 5, body, 0, unroll=True)
        return 0

    lax.fori_loop(0, VPT // 5, chunk, 0)
    pltpu.sync_copy(cnt, counts_hbm.at[pl.ds(_m8(w * 32), 32)])

    v0 = z
    v1 = z
    for b in range(32):
        s = _ssum(cnt[b, pl.ds(0, 16)])
        sel = jnp.where(iota == (b & 15), ones, z)
        if b < 16:
            v0 = v0 + s * sel
        else:
            v1 = v1 + s * sel
    cntc[pl.ds(0, 16)] = v0
    cntc[pl.ds(16, 16)] = v1
    pltpu.sync_copy(cntc, cc_hbm.at[pl.ds(_m8(w * 32), 32)])


_hist = pl.kernel(
    _hist_body,
    out_type=(
        jax.ShapeDtypeStruct((NW * 32, 16), jnp.int32),
        jax.ShapeDtypeStruct((NW * 32,), jnp.int32),
    ),
    mesh=_mesh(),
    compiler_params=pltpu.CompilerParams(use_tc_tiling_on_sc=False),
    scratch_types=[
        pltpu.VMEM((EPT,), jnp.int32),
        pltpu.VMEM((32, 16), jnp.int32),
        pltpu.VMEM((32,), jnp.int32),
    ],
)


def _bin_body(src_hbm, dst_hbm, counts_hbm, perm_hbm,
              sbuf, dbuf, cbuf, base, stg_pos, stg_dat, sem):
    w = _wid()
    pltpu.sync_copy(src_hbm.at[pl.ds(_m8(w * EPT), EPT)], sbuf)
    pltpu.sync_copy(dst_hbm.at[pl.ds(_m8(w * EPT), EPT)], dbuf)
    pltpu.sync_copy(counts_hbm, cbuf)
    iota = _iota()
    zi = jnp.zeros((16,), jnp.int32)

    def bb(b, gstart):
        def wacc(wp, carry):
            t, p_, m_ = carry
            v = cbuf[wp * 32 + b, pl.ds(0, 16)]
            fp = jnp.where(wp < w, 1, 0)
            fm = jnp.where(wp == w, 1, 0)
            return (t + v, p_ + v * fp, m_ + v * fm)

        totv, priorv, myv = lax.fori_loop(0, NW, wacc, (zi, zi, zi))
        inc = myv
        for dsh in (1, 2, 4, 8):
            g = _take(inc, jnp.maximum(iota - dsh, 0))
            inc = inc + jnp.where(iota >= dsh, g, zi)
        base[b, pl.ds(0, 16)] = gstart + _ssum(priorv) + (inc - myv)
        return gstart + _ssum(totv)

    lax.fori_loop(0, 32, bb, jnp.int32(0))
    onehots = [jnp.where(iota == k, 1, 0) for k in range(16)]

    def chunk(c, _):
        slot = lax.rem(c, 2)
        soff = _m8(slot * 160)

        @pl.when(c >= 2)
        def _wait_prev():
            pltpu.make_async_copy(
                stg_dat.at[pl.ds(soff, 160)],
                perm_hbm.at[stg_pos.at[slot]], sem).wait()

        def vbody(v, _):
            k = _m8(c * 80 + v * 16)
            s = sbuf[pl.ds(k, 16)]
            d = dbuf[pl.ds(k, 16)]
            b = _bucket(d)
            pos = jnp.zeros((16,), jnp.int32)
            for t in range(32):
                rowv = base[t, pl.ds(0, 16)]
                mi = jnp.where(b == t, 1, 0)
                pos = pos + rowv * mi
                base[t, pl.ds(0, 16)] = rowv + mi
            stg_pos[slot, pl.ds(v * 16, 16)] = pos
            stg_pos[slot, pl.ds(80 + v * 16, 16)] = pos + E
            stg_dat[pl.ds(_m8(soff + v * 16), 16)] = s
            stg_dat[pl.ds(_m8(soff + 80 + v * 16), 16)] = d
            return 0

        lax.fori_loop(0, 5, vbody, 0)
        pltpu.async_copy(
            stg_dat.at[pl.ds(soff, 160)],
            perm_hbm.at[stg_pos.at[slot]], sem)
        return 0

    lax.fori_loop(0, EPT // 80, chunk, 0)
    for sl in (0, 1):
        pltpu.make_async_copy(
            stg_dat.at[pl.ds(sl * 160, 160)],
            perm_hbm.at[stg_pos.at[sl]], sem).wait()


_bin = pl.kernel(
    _bin_body,
    out_type=jax.ShapeDtypeStruct((2 * E,), jnp.int32),
    mesh=_mesh(),
    compiler_params=pltpu.CompilerParams(use_tc_tiling_on_sc=False),
    scratch_types=[
        pltpu.VMEM((EPT,), jnp.int32),
        pltpu.VMEM((EPT,), jnp.int32),
        pltpu.VMEM((NW * 32, 16), jnp.int32),
        pltpu.VMEM((32, 16), jnp.int32),
        pltpu.VMEM((2, 160), jnp.int32),
        pltpu.VMEM((320,), jnp.int32),
        pltpu.SemaphoreType.DMA,
    ],
)


def _gx_body(a_hbm, bn_hbm, srcp, dstp, counts_hbm, x_hbm,
             cbuf, sidx, didx, arows, brows, semi, semg):
    w = _wid()
    pltpu.sync_copy(counts_hbm, cbuf)
    start, nedges = _prefix_counts(cbuf, w)
    cstart = _m8(jnp.bitwise_and(start, -8))
    cend = _m8(jnp.bitwise_and(start + nedges + 7, -8))
    nch = (cend - cstart + SCHUNK - 1) // SCHUNK

    def offof(c):
        return _m8(jnp.maximum(0, jnp.minimum(cstart + c * SCHUNK,
                                              cend - SCHUNK)))

    def issue_idx(c):
        slot = lax.rem(c, 2)
        soff = _m8(slot * SCHUNK)
        off = offof(c)
        pltpu.async_copy(srcp.at[pl.ds(off, SCHUNK)],
                         sidx.at[pl.ds(soff, SCHUNK)], semi)
        pltpu.async_copy(dstp.at[pl.ds(off, SCHUNK)],
                         didx.at[pl.ds(soff, SCHUNK)], semi)

    @pl.when(nch > 0)
    def _prologue():
        issue_idx(0)

    def chunk(c, _):
        slot = lax.rem(c, 2)
        soff = _m8(slot * SCHUNK)
        pltpu.make_async_copy(
            srcp.at[pl.ds(0, SCHUNK)],
            sidx.at[pl.ds(0, SCHUNK)], semi).wait()
        pltpu.make_async_copy(
            srcp.at[pl.ds(0, SCHUNK)],
            didx.at[pl.ds(0, SCHUNK)], semi).wait()
        for hblk in range(SCHUNK // 128):
            pltpu.async_copy(
                a_hbm.at[sidx.at[pl.ds(_m8(soff + hblk * 128), 128)]],
                arows.at[pl.ds(_m8(soff + hblk * 128), 128)], semg)
            pltpu.async_copy(
                bn_hbm.at[didx.at[pl.ds(_m8(soff + hblk * 128), 128)]],
                brows.at[pl.ds(_m8(soff + hblk * 128), 128)], semg)

        @pl.when(c + 1 < nch)
        def _next():
            issue_idx(c + 1)

        for _ in range(2 * (SCHUNK // 128)):
            pltpu.make_async_copy(
                a_hbm.at[pl.ds(0, 128)],
                arows.at[pl.ds(0, 128)], semg).wait()

        @plsc.parallel_loop(0, SCHUNK, unroll=4)
        def _combine(r):
            arows[soff + r, pl.ds(0, 16)] = (
                arows[soff + r, pl.ds(0, 16)] + brows[soff + r, pl.ds(0, 16)])
            arows[soff + r, pl.ds(16, 16)] = (
                arows[soff + r, pl.ds(16, 16)] + brows[soff + r, pl.ds(16, 16)])

        pltpu.sync_copy(arows.at[pl.ds(soff, SCHUNK)],
                        x_hbm.at[pl.ds(offof(c), SCHUNK)])
        return 0

    lax.fori_loop(0, nch, chunk, 0)


_gx = pl.kernel(
    _gx_body,
    out_type=jax.ShapeDtypeStruct((E, H), jnp.float32),
    mesh=_mesh(),
    compiler_params=pltpu.CompilerParams(use_tc_tiling_on_sc=False),
    scratch_types=[
        pltpu.VMEM((NW * 32,), jnp.int32),
        pltpu.VMEM((2 * SCHUNK,), jnp.int32),
        pltpu.VMEM((2 * SCHUNK,), jnp.int32),
        pltpu.VMEM((2 * SCHUNK, H), jnp.float32),
        pltpu.VMEM((2 * SCHUNK, H), jnp.float32),
        pltpu.SemaphoreType.DMA,
        pltpu.SemaphoreType.DMA,
    ],
)


def _smax_body(final, m_hbm, dstp, counts_hbm, batch, out_hbm,
               cbuf, acc, mrows, didx, widx, gacc, bbuf, semi, semm):
    w = _wid()
    iota = _iota()
    pltpu.sync_copy(counts_hbm, cbuf)

    def _bst_at(k):
        return jnp.minimum((k * (1 << BSHIFT) + BMULT - 1) // BMULT, N)

    nstart = _bst_at(w)
    nend = _bst_at(w + 1)
    nrows = nend - nstart
    z16 = jnp.zeros((16,), jnp.float32)

    @plsc.parallel_loop(0, ACCROWS, unroll=8)
    def _zero(r):
        acc[r, pl.ds(0, 16)] = z16
        acc[r, pl.ds(16, 16)] = z16

    start, nedges = _prefix_counts(cbuf, w)
    cstart = _m8(jnp.bitwise_and(start, -8))
    cend = _m8(jnp.bitwise_and(start + nedges + 7, -8))
    nch = (cend - cstart + MCHUNK - 1) // MCHUNK

    def offof(c):
        return _m8(jnp.maximum(0, jnp.minimum(cstart + c * MCHUNK,
                                              cend - MCHUNK)))

    def issue(c):
        slot = lax.rem(c, 2)
        soff = _m8(slot * MCHUNK)
        off = offof(c)
        pltpu.async_copy(dstp.at[pl.ds(off, MCHUNK)],
                         didx.at[pl.ds(soff, MCHUNK)], semi)
        pltpu.async_copy(m_hbm.at[pl.ds(off, MCHUNK)],
                         mrows.at[pl.ds(soff, MCHUNK)], semm)

    @pl.when(nch > 0)
    def _prologue():
        issue(0)

    def chunk(c, _):
        slot = lax.rem(c, 2)
        soff = _m8(slot * MCHUNK)
        pltpu.make_async_copy(
            dstp.at[pl.ds(0, MCHUNK)],
            didx.at[pl.ds(0, MCHUNK)], semi).wait()
        pltpu.make_async_copy(
            m_hbm.at[pl.ds(0, MCHUNK)],
            mrows.at[pl.ds(0, MCHUNK)], semm).wait()

        @pl.when(c + 1 < nch)
        def _next():
            issue(c + 1)

        def eb(v, _):
            lv = didx[pl.ds(_m8(soff + v * 16), 16)] - nstart
            valid = (lv >= 0) & (lv < nrows)
            lv = jnp.where(valid, lv, DUMROW)
            for k in range(16):
                l = lv[k]
                e = soff + v * 16 + k
                acc[l, pl.ds(0, 16)] = jnp.maximum(
                    acc[l, pl.ds(0, 16)], mrows[e, pl.ds(0, 16)])
                acc[l, pl.ds(16, 16)] = jnp.maximum(
                    acc[l, pl.ds(16, 16)], mrows[e, pl.ds(16, 16)])
            return 0

        lax.fori_loop(0, MCHUNK // 16, eb, 0)
        return 0

    lax.fori_loop(0, nch, chunk, 0)

    if not final:
        nwch = (nrows + 127) // 128

        def wb(cc, _):
            roff = _m8(cc * 128)

            def ib(v, _):
                gi = nstart + roff + v * 16 + iota
                widx[pl.ds(v * 16, 16)] = jnp.where(
                    gi < nend, gi, N + jnp.bitwise_and(gi, NPAD - 1))
                return 0

            lax.fori_loop(0, 8, ib, 0, unroll=True)
            pltpu.async_copy(
                acc.at[pl.ds(roff, 128)], out_hbm.at[widx], semm).wait()
            return 0

        lax.fori_loop(0, nwch, wb, 0)
    else:
        @plsc.parallel_loop(0, G + 1)
        def _gz(r):
            gacc[r, pl.ds(0, 16)] = z16
            gacc[r, pl.ds(16, 16)] = z16

        bs_al = _m8(jnp.bitwise_and(nstart, -8))
        be_al = _m8(jnp.bitwise_and(nend + 7, -8))
        nbch = (be_al - bs_al + MCHUNK - 1) // MCHUNK

        def gb(cc, _):
            boff = _m8(jnp.maximum(
                0, jnp.minimum(bs_al + cc * MCHUNK, be_al - MCHUNK)))
            pltpu.sync_copy(batch.at[pl.ds(boff, MCHUNK)], bbuf)

            def nb(v, _):
                l = boff + v * 16 + iota - nstart
                valid = (l >= 0) & (l < nrows)
                gv = jnp.where(valid, bbuf[pl.ds(v * 16, 16)], G)
                lv = jnp.where(valid, l, DUMROW)
                for k in range(16):
                    g = gv[k]
                    lk = lv[k]
                    gacc[g, pl.ds(0, 16)] = jnp.maximum(
                        gacc[g, pl.ds(0, 16)], acc[lk, pl.ds(0, 16)])
                    gacc[g, pl.ds(16, 16)] = jnp.maximum(
                        gacc[g, pl.ds(16, 16)], acc[lk, pl.ds(16, 16)])
                return 0

            lax.fori_loop(0, MCHUNK // 16, nb, 0)
            return 0

        lax.fori_loop(0, nbch, gb, 0)
        pltpu.sync_copy(gacc.at[pl.ds(0, G)], out_hbm.at[w])


def _make_smax(final):
    out_type = (jax.ShapeDtypeStruct((NW, G, H), jnp.float32) if final
                else jax.ShapeDtypeStruct((N + NPAD, H), jnp.float32))
    return pl.kernel(
        functools.partial(_smax_body, final),
        out_type=out_type,
        mesh=_mesh(),
        compiler_params=pltpu.CompilerParams(use_tc_tiling_on_sc=False),
        scratch_types=[
            pltpu.VMEM((NW * 32,), jnp.int32),
            pltpu.VMEM((ACCROWS, H), jnp.float32),
            pltpu.VMEM((2 * MCHUNK, H), jnp.float32),
            pltpu.VMEM((2 * MCHUNK,), jnp.int32),
            pltpu.VMEM((128,), jnp.int32),
            pltpu.VMEM((G + 1, H), jnp.float32),
            pltpu.VMEM((MCHUNK,), jnp.int32),
            pltpu.SemaphoreType.DMA,
            pltpu.SemaphoreType.DMA,
        ],
    )


_smax1 = _make_smax(False)
_smax2 = _make_smax(True)


def _nt1_body(pos_ref, w_ref, b_ref, a_ref, bn_ref):
    p = pos_ref[...]
    wfull = w_ref[...]
    wh = wfull[0:3]
    wp = wfull[3:6]
    p0, p1, p2 = p[:, 0:1], p[:, 1:2], p[:, 2:3]
    bp = p0 * wp[0] + p1 * wp[1] + p2 * wp[2]
    ah = p0 * wh[0] + p1 * wh[1] + p2 * wh[2]
    a_ref[...] = ah + bp + b_ref[...]
    bn_ref[...] = -bp


_nt1 = pl.pallas_call(
    _nt1_body,
    grid=(N // NBLK,),
    in_specs=[
        pl.BlockSpec((NBLK, 3), lambda i: (i, 0)),
        pl.BlockSpec((6, H), lambda i: (0, 0)),
        pl.BlockSpec((H,), lambda i: (0,)),
    ],
    out_specs=[
        pl.BlockSpec((NBLK, H), lambda i: (i, 0)),
        pl.BlockSpec((NBLK, H), lambda i: (i, 0)),
    ],
    out_shape=[
        jax.ShapeDtypeStruct((N, H), jnp.float32),
        jax.ShapeDtypeStruct((N, H), jnp.float32),
    ],
)


def _nt2_body(h_ref, pos_ref, w_ref, b_ref, a_ref, bn_ref):
    h = h_ref[...]
    p = pos_ref[...]
    wfull = w_ref[...]
    wh = wfull[0:H]
    wp = wfull[H:H + 3]
    bp = p[:, 0:1] * wp[0] + p[:, 1:2] * wp[1] + p[:, 2:3] * wp[2]
    a_ref[...] = (jnp.dot(h, wh, preferred_element_type=jnp.float32)
                  + bp + b_ref[...])
    bn_ref[...] = -bp


_nt2 = pl.pallas_call(
    _nt2_body,
    grid=(N // NBLK,),
    in_specs=[
        pl.BlockSpec((NBLK, H), lambda i: (i, 0)),
        pl.BlockSpec((NBLK, 3), lambda i: (i, 0)),
        pl.BlockSpec((H + 3, H), lambda i: (0, 0)),
        pl.BlockSpec((H,), lambda i: (0,)),
    ],
    out_specs=[
        pl.BlockSpec((NBLK, H), lambda i: (i, 0)),
        pl.BlockSpec((NBLK, H), lambda i: (i, 0)),
    ],
    out_shape=[
        jax.ShapeDtypeStruct((N, H), jnp.float32),
        jax.ShapeDtypeStruct((N, H), jnp.float32),
    ],
)


def _mlp_body(x_ref, w_ref, b_ref, m_ref):
    y = jnp.maximum(x_ref[...], 0.0)
    m_ref[...] = (jnp.dot(y, w_ref[...], preferred_element_type=jnp.float32)
                  + b_ref[...])


_mlp = pl.pallas_call(
    _mlp_body,
    grid=(E // NBLK,),
    in_specs=[
        pl.BlockSpec((NBLK, H), lambda i: (i, 0)),
        pl.BlockSpec((H, H), lambda i: (0, 0)),
        pl.BlockSpec((H,), lambda i: (0,)),
    ],
    out_specs=pl.BlockSpec((NBLK, H), lambda i: (i, 0)),
    out_shape=jax.ShapeDtypeStruct((E, H), jnp.float32),
)


def _fin_body(par_ref, w_ref, b_ref, o_ref):
    red = jnp.max(par_ref[...], axis=0)
    o_ref[...] = (jnp.dot(red, w_ref[...], preferred_element_type=jnp.float32)
                  + b_ref[...])


_fin = pl.pallas_call(
    _fin_body,
    out_shape=jax.ShapeDtypeStruct((G, 1), jnp.float32),
)


def kernel(pos, edge_index, batch, W1a, b1a, W1b, b1b, W2a, b2a, W2b, b2b,
           Wout, bout):
    src = edge_index[0]
    dst = edge_index[1]
    counts, countsc = _hist(dst)
    perm = _bin(src, dst, counts)
    srcp = perm[:E]
    dstp = perm[E:]
    a1, b1n = _nt1(pos, W1a, b1a)
    x1 = _gx(a1, b1n, srcp, dstp, countsc)
    m1 = _mlp(x1, W1b, b1b)
    h1 = _smax1(m1, dstp, countsc, batch)[:N]
    a2, b2n = _nt2(h1, pos, W2a, b2a)
    x2 = _gx(a2, b2n, srcp, dstp, countsc)
    m2 = _mlp(x2, W2b, b2b)
    partials = _smax2(m2, dstp, countsc, batch)
    return _fin(partials, Wout, bout)


# speedup vs baseline: 1.0004x; 1.0004x over previous
import functools

import jax
import jax.numpy as jnp
import numpy as np
from jax import lax
from jax.experimental import pallas as pl
from jax.experimental.pallas import tpu as pltpu
from jax.experimental.pallas import tpu_sc as plsc

N = 100000
E = 1600000
H = 32
G = 64
NC = 2
NS = 16
NW = NC * NS
EPT = E // NW
VPT = EPT // 16
BMULT = 41
BSHIFT = 17
MAXROWS = 3197
ACCROWS = 3328
DUMROW = ACCROWS - 1
NPAD = 128
SCHUNK = 256
MCHUNK = 256
NBLK = 4000

_BSTARTS_NP = np.minimum(
    (np.arange(48, dtype=np.int64) * (1 << BSHIFT) + BMULT - 1) // BMULT, N
).astype(np.int32)


def _mesh():
    return plsc.VectorSubcoreMesh(core_axis_name="c", subcore_axis_name="s")


def _wid():
    return lax.axis_index("s") * NC + lax.axis_index("c")


def _take(x, i):
    return x.at[i].get(mode="promise_in_bounds")


def _iota():
    return lax.iota(jnp.int32, 16)


def _bucket(d):
    return (d * BMULT) >> BSHIFT


def _m8(x):
    return pl.multiple_of(x, 8)


def _vsum(v):
    iota = _iota()
    for d in (1, 2, 4, 8):
        v = v + _take(v, jnp.bitwise_xor(iota, d))
    return v


def _ssum(v):
    return _vsum(v)[0]


def _prefix_counts(cbuf, w):
    iota = _iota()
    zi = jnp.zeros((16,), jnp.int32)

    def pf(p, carry):
        s0, s1 = carry
        v = cbuf[pl.ds(_m8(p * 16), 16)]
        b = jnp.bitwise_and(p * 16 + iota, 31)
        s0 = s0 + jnp.where(b < w, v, zi)
        s1 = s1 + jnp.where(b == w, v, zi)
        return (s0, s1)

    s0, s1 = lax.fori_loop(0, NW * 32 // 16, pf, (zi, zi))
    return _ssum(s0), _ssum(s1)


def _hist_body(dst_hbm, counts_hbm, cc_hbm, dbuf, cnt, cntc):
    w = _wid()
    pltpu.sync_copy(dst_hbm.at[pl.ds(_m8(w * EPT), EPT)], dbuf)
    iota = _iota()
    z = jnp.zeros((16,), jnp.int32)
    ones = jnp.ones((16,), jnp.int32)
    for p in range(32):
        cnt[p, pl.ds(0, 16)] = z
    onehots = [jnp.where(iota == k, 1, 0) for k in range(16)]

    def chunk(c, _):
        def body(v, _):
            d = dbuf[pl.ds(_m8(c * 80 + v * 16), 16)]
            b = _bucket(d)
            for k in range(16):
                bk = b[k]
                cnt[bk, pl.ds(0, 16)] = cnt[bk, pl.ds(0, 16)] + onehots[k]
            return 0

        lax.fori_loop(0, 5, body, 0, unroll=True)
        return 0

    lax.fori_loop(0, VPT // 5, chunk, 0)
    pltpu.sync_copy(cnt, counts_hbm.at[pl.ds(_m8(w * 32), 32)])

    v0 = z
    v1 = z
    for b in range(32):
        s = _ssum(cnt[b, pl.ds(0, 16)])
        sel = jnp.where(iota == (b & 15), ones, z)
        if b < 16:
            v0 = v0 + s * sel
        else:
            v1 = v1 + s * sel
    cntc[pl.ds(0, 16)] = v0
    cntc[pl.ds(16, 16)] = v1
    pltpu.sync_copy(cntc, cc_hbm.at[pl.ds(_m8(w * 32), 32)])


_hist = pl.kernel(
    _hist_body,
    out_type=(
        jax.ShapeDtypeStruct((NW * 32, 16), jnp.int32),
        jax.ShapeDtypeStruct((NW * 32,), jnp.int32),
    ),
    mesh=_mesh(),
    compiler_params=pltpu.CompilerParams(use_tc_tiling_on_sc=False),
    scratch_types=[
        pltpu.VMEM((EPT,), jnp.int32),
        pltpu.VMEM((32, 16), jnp.int32),
        pltpu.VMEM((32,), jnp.int32),
    ],
)


def _bin_body(src_hbm, dst_hbm, counts_hbm, perm_hbm,
              sbuf, dbuf, cbuf, base, stg_pos, stg_dat, sem):
    w = _wid()
    pltpu.sync_copy(src_hbm.at[pl.ds(_m8(w * EPT), EPT)], sbuf)
    pltpu.sync_copy(dst_hbm.at[pl.ds(_m8(w * EPT), EPT)], dbuf)
    pltpu.sync_copy(counts_hbm, cbuf)
    iota = _iota()
    zi = jnp.zeros((16,), jnp.int32)

    def bb(b, gstart):
        def wacc(wp, carry):
            t, p_, m_ = carry
            v = cbuf[wp * 32 + b, pl.ds(0, 16)]
            fp = jnp.where(wp < w, 1, 0)
            fm = jnp.where(wp == w, 1, 0)
            return (t + v, p_ + v * fp, m_ + v * fm)

        totv, priorv, myv = lax.fori_loop(0, NW, wacc, (zi, zi, zi))
        inc = myv
        for dsh in (1, 2, 4, 8):
            g = _take(inc, jnp.maximum(iota - dsh, 0))
            inc = inc + jnp.where(iota >= dsh, g, zi)
        base[b, pl.ds(0, 16)] = gstart + _ssum(priorv) + (inc - myv)
        return gstart + _ssum(totv)

    lax.fori_loop(0, 32, bb, jnp.int32(0))
    onehots = [jnp.where(iota == k, 1, 0) for k in range(16)]

    def chunk(c, _):
        slot = lax.rem(c, 8)
        soff = _m8(slot * 160)

        @pl.when(c >= 8)
        def _wait_prev():
            pltpu.make_async_copy(
                stg_dat.at[pl.ds(soff, 160)],
                perm_hbm.at[stg_pos.at[slot]], sem).wait()

        def vbody(v, _):
            k = _m8(c * 80 + v * 16)
            s = sbuf[pl.ds(k, 16)]
            d = dbuf[pl.ds(k, 16)]
            b = _bucket(d)
            pos = jnp.zeros((16,), jnp.int32)
            for t in range(32):
                rowv = base[t, pl.ds(0, 16)]
                mi = jnp.where(b == t, 1, 0)
                pos = pos + rowv * mi
                base[t, pl.ds(0, 16)] = rowv + mi
            stg_pos[slot, pl.ds(v * 16, 16)] = pos
            stg_pos[slot, pl.ds(80 + v * 16, 16)] = pos + E
            stg_dat[pl.ds(_m8(soff + v * 16), 16)] = s
            stg_dat[pl.ds(_m8(soff + 80 + v * 16), 16)] = d
            return 0

        lax.fori_loop(0, 5, vbody, 0)
        pltpu.async_copy(
            stg_dat.at[pl.ds(soff, 160)],
            perm_hbm.at[stg_pos.at[slot]], sem)
        return 0

    lax.fori_loop(0, EPT // 80, chunk, 0)
    for sl in range(8):
        pltpu.make_async_copy(
            stg_dat.at[pl.ds(sl * 160, 160)],
            perm_hbm.at[stg_pos.at[sl]], sem).wait()


_bin = pl.kernel(
    _bin_body,
    out_type=jax.ShapeDtypeStruct((2 * E,), jnp.int32),
    mesh=_mesh(),
    compiler_params=pltpu.CompilerParams(use_tc_tiling_on_sc=False),
    scratch_types=[
        pltpu.VMEM((EPT,), jnp.int32),
        pltpu.VMEM((EPT,), jnp.int32),
        pltpu.VMEM((NW * 32, 16), jnp.int32),
        pltpu.VMEM((32, 16), jnp.int32),
        pltpu.VMEM((8, 160), jnp.int32),
        pltpu.VMEM((1280,), jnp.int32),
        pltpu.SemaphoreType.DMA,
    ],
)


def _gx_body(a_hbm, bn_hbm, srcp, dstp, counts_hbm, x_hbm,
             cbuf, sidx, didx, arows, brows, semi, semg):
    w = _wid()
    pltpu.sync_copy(counts_hbm, cbuf)
    start, nedges = _prefix_counts(cbuf, w)
    cstart = _m8(jnp.bitwise_and(start, -8))
    cend = _m8(jnp.bitwise_and(start + nedges + 7, -8))
    nch = (cend - cstart + SCHUNK - 1) // SCHUNK

    def offof(c):
        return _m8(jnp.maximum(0, jnp.minimum(cstart + c * SCHUNK,
                                              cend - SCHUNK)))

    def issue_idx(c):
        slot = lax.rem(c, 2)
        soff = _m8(slot * SCHUNK)
        off = offof(c)
        pltpu.async_copy(srcp.at[pl.ds(off, SCHUNK)],
                         sidx.at[pl.ds(soff, SCHUNK)], semi)
        pltpu.async_copy(dstp.at[pl.ds(off, SCHUNK)],
                         didx.at[pl.ds(soff, SCHUNK)], semi)

    @pl.when(nch > 0)
    def _prologue():
        issue_idx(0)

    def chunk(c, _):
        slot = lax.rem(c, 2)
        soff = _m8(slot * SCHUNK)
        pltpu.make_async_copy(
            srcp.at[pl.ds(0, SCHUNK)],
            sidx.at[pl.ds(0, SCHUNK)], semi).wait()
        pltpu.make_async_copy(
            srcp.at[pl.ds(0, SCHUNK)],
            didx.at[pl.ds(0, SCHUNK)], semi).wait()
        for hblk in range(SCHUNK // 128):
            pltpu.async_copy(
                a_hbm.at[sidx.at[pl.ds(_m8(soff + hblk * 128), 128)]],
                arows.at[pl.ds(_m8(soff + hblk * 128), 128)], semg)
            pltpu.async_copy(
                bn_hbm.at[didx.at[pl.ds(_m8(soff + hblk * 128), 128)]],
                brows.at[pl.ds(_m8(soff + hblk * 128), 128)], semg)

        @pl.when(c + 1 < nch)
        def _next():
            issue_idx(c + 1)

        for _ in range(2 * (SCHUNK // 128)):
            pltpu.make_async_copy(
                a_hbm.at[pl.ds(0, 128)],
                arows.at[pl.ds(0, 128)], semg).wait()

        @plsc.parallel_loop(0, SCHUNK, unroll=4)
        def _combine(r):
            arows[soff + r, pl.ds(0, 16)] = (
                arows[soff + r, pl.ds(0, 16)] + brows[soff + r, pl.ds(0, 16)])
            arows[soff + r, pl.ds(16, 16)] = (
                arows[soff + r, pl.ds(16, 16)] + brows[soff + r, pl.ds(16, 16)])

        pltpu.sync_copy(arows.at[pl.ds(soff, SCHUNK)],
                        x_hbm.at[pl.ds(offof(c), SCHUNK)])
        return 0

    lax.fori_loop(0, nch, chunk, 0)


_gx = pl.kernel(
    _gx_body,
    out_type=jax.ShapeDtypeStruct((E, H), jnp.float32),
    mesh=_mesh(),
    compiler_params=pltpu.CompilerParams(use_tc_tiling_on_sc=False),
    scratch_types=[
        pltpu.VMEM((NW * 32,), jnp.int32),
        pltpu.VMEM((2 * SCHUNK,), jnp.int32),
        pltpu.VMEM((2 * SCHUNK,), jnp.int32),
        pltpu.VMEM((2 * SCHUNK, H), jnp.float32),
        pltpu.VMEM((2 * SCHUNK, H), jnp.float32),
        pltpu.SemaphoreType.DMA,
        pltpu.SemaphoreType.DMA,
    ],
)


def _smax_body(final, m_hbm, dstp, counts_hbm, batch, out_hbm,
               cbuf, acc, mrows, didx, widx, gacc, bbuf, semi, semm):
    w = _wid()
    iota = _iota()
    pltpu.sync_copy(counts_hbm, cbuf)

    def _bst_at(k):
        return jnp.minimum((k * (1 << BSHIFT) + BMULT - 1) // BMULT, N)

    nstart = _bst_at(w)
    nend = _bst_at(w + 1)
    nrows = nend - nstart
    z16 = jnp.zeros((16,), jnp.float32)

    @plsc.parallel_loop(0, ACCROWS, unroll=8)
    def _zero(r):
        acc[r, pl.ds(0, 16)] = z16
        acc[r, pl.ds(16, 16)] = z16

    start, nedges = _prefix_counts(cbuf, w)
    cstart = _m8(jnp.bitwise_and(start, -8))
    cend = _m8(jnp.bitwise_and(start + nedges + 7, -8))
    nch = (cend - cstart + MCHUNK - 1) // MCHUNK

    def offof(c):
        return _m8(jnp.maximum(0, jnp.minimum(cstart + c * MCHUNK,
                                              cend - MCHUNK)))

    def issue(c):
        slot = lax.rem(c, 2)
        soff = _m8(slot * MCHUNK)
        off = offof(c)
        pltpu.async_copy(dstp.at[pl.ds(off, MCHUNK)],
                         didx.at[pl.ds(soff, MCHUNK)], semi)
        pltpu.async_copy(m_hbm.at[pl.ds(off, MCHUNK)],
                         mrows.at[pl.ds(soff, MCHUNK)], semm)

    @pl.when(nch > 0)
    def _prologue():
        issue(0)

    def chunk(c, _):
        slot = lax.rem(c, 2)
        soff = _m8(slot * MCHUNK)
        pltpu.make_async_copy(
            dstp.at[pl.ds(0, MCHUNK)],
            didx.at[pl.ds(0, MCHUNK)], semi).wait()
        pltpu.make_async_copy(
            m_hbm.at[pl.ds(0, MCHUNK)],
            mrows.at[pl.ds(0, MCHUNK)], semm).wait()

        @pl.when(c + 1 < nch)
        def _next():
            issue(c + 1)

        def eb(v, _):
            lv = didx[pl.ds(_m8(soff + v * 16), 16)] - nstart
            valid = (lv >= 0) & (lv < nrows)
            lv = jnp.where(valid, lv, DUMROW)
            for k in range(16):
                l = lv[k]
                e = soff + v * 16 + k
                acc[l, pl.ds(0, 16)] = jnp.maximum(
                    acc[l, pl.ds(0, 16)], mrows[e, pl.ds(0, 16)])
                acc[l, pl.ds(16, 16)] = jnp.maximum(
                    acc[l, pl.ds(16, 16)], mrows[e, pl.ds(16, 16)])
            return 0

        lax.fori_loop(0, MCHUNK // 16, eb, 0)
        return 0

    lax.fori_loop(0, nch, chunk, 0)

    if not final:
        nwch = (nrows + 127) // 128

        def wb(cc, _):
            roff = _m8(cc * 128)

            def ib(v, _):
                gi = nstart + roff + v * 16 + iota
                widx[pl.ds(v * 16, 16)] = jnp.where(
                    gi < nend, gi, N + jnp.bitwise_and(gi, NPAD - 1))
                return 0

            lax.fori_loop(0, 8, ib, 0, unroll=True)
            pltpu.async_copy(
                acc.at[pl.ds(roff, 128)], out_hbm.at[widx], semm).wait()
            return 0

        lax.fori_loop(0, nwch, wb, 0)
    else:
        @plsc.parallel_loop(0, G + 1)
        def _gz(r):
            gacc[r, pl.ds(0, 16)] = z16
            gacc[r, pl.ds(16, 16)] = z16

        bs_al = _m8(jnp.bitwise_and(nstart, -8))
        be_al = _m8(jnp.bitwise_and(nend + 7, -8))
        nbch = (be_al - bs_al + MCHUNK - 1) // MCHUNK

        def gb(cc, _):
            boff = _m8(jnp.maximum(
                0, jnp.minimum(bs_al + cc * MCHUNK, be_al - MCHUNK)))
            pltpu.sync_copy(batch.at[pl.ds(boff, MCHUNK)], bbuf)

            def nb(v, _):
                l = boff + v * 16 + iota - nstart
                valid = (l >= 0) & (l < nrows)
                gv = jnp.where(valid, bbuf[pl.ds(v * 16, 16)], G)
                lv = jnp.where(valid, l, DUMROW)
                for k in range(16):
                    g = gv[k]
                    lk = lv[k]
                    gacc[g, pl.ds(0, 16)] = jnp.maximum(
                        gacc[g, pl.ds(0, 16)], acc[lk, pl.ds(0, 16)])
                    gacc[g, pl.ds(16, 16)] = jnp.maximum(
                        gacc[g, pl.ds(16, 16)], acc[lk, pl.ds(16, 16)])
                return 0

            lax.fori_loop(0, MCHUNK // 16, nb, 0)
            return 0

        lax.fori_loop(0, nbch, gb, 0)
        pltpu.sync_copy(gacc.at[pl.ds(0, G)], out_hbm.at[w])


def _make_smax(final):
    out_type = (jax.ShapeDtypeStruct((NW, G, H), jnp.float32) if final
                else jax.ShapeDtypeStruct((N + NPAD, H), jnp.float32))
    return pl.kernel(
        functools.partial(_smax_body, final),
        out_type=out_type,
        mesh=_mesh(),
        compiler_params=pltpu.CompilerParams(use_tc_tiling_on_sc=False),
        scratch_types=[
            pltpu.VMEM((NW * 32,), jnp.int32),
            pltpu.VMEM((ACCROWS, H), jnp.float32),
            pltpu.VMEM((2 * MCHUNK, H), jnp.float32),
            pltpu.VMEM((2 * MCHUNK,), jnp.int32),
            pltpu.VMEM((128,), jnp.int32),
            pltpu.VMEM((G + 1, H), jnp.float32),
            pltpu.VMEM((MCHUNK,), jnp.int32),
            pltpu.SemaphoreType.DMA,
            pltpu.SemaphoreType.DMA,
        ],
    )


_smax1 = _make_smax(False)
_smax2 = _make_smax(True)


def _nt1_body(pos_ref, w_ref, b_ref, a_ref, bn_ref):
    p = pos_ref[...]
    wfull = w_ref[...]
    wh = wfull[0:3]
    wp = wfull[3:6]
    p0, p1, p2 = p[:, 0:1], p[:, 1:2], p[:, 2:3]
    bp = p0 * wp[0] + p1 * wp[1] + p2 * wp[2]
    ah = p0 * wh[0] + p1 * wh[1] + p2 * wh[2]
    a_ref[...] = ah + bp + b_ref[...]
    bn_ref[...] = -bp


_nt1 = pl.pallas_call(
    _nt1_body,
    grid=(N // NBLK,),
    in_specs=[
        pl.BlockSpec((NBLK, 3), lambda i: (i, 0)),
        pl.BlockSpec((6, H), lambda i: (0, 0)),
        pl.BlockSpec((H,), lambda i: (0,)),
    ],
    out_specs=[
        pl.BlockSpec((NBLK, H), lambda i: (i, 0)),
        pl.BlockSpec((NBLK, H), lambda i: (i, 0)),
    ],
    out_shape=[
        jax.ShapeDtypeStruct((N, H), jnp.float32),
        jax.ShapeDtypeStruct((N, H), jnp.float32),
    ],
)


def _nt2_body(h_ref, pos_ref, w_ref, b_ref, a_ref, bn_ref):
    h = h_ref[...]
    p = pos_ref[...]
    wfull = w_ref[...]
    wh = wfull[0:H]
    wp = wfull[H:H + 3]
    bp = p[:, 0:1] * wp[0] + p[:, 1:2] * wp[1] + p[:, 2:3] * wp[2]
    a_ref[...] = (jnp.dot(h, wh, preferred_element_type=jnp.float32)
                  + bp + b_ref[...])
    bn_ref[...] = -bp


_nt2 = pl.pallas_call(
    _nt2_body,
    grid=(N // NBLK,),
    in_specs=[
        pl.BlockSpec((NBLK, H), lambda i: (i, 0)),
        pl.BlockSpec((NBLK, 3), lambda i: (i, 0)),
        pl.BlockSpec((H + 3, H), lambda i: (0, 0)),
        pl.BlockSpec((H,), lambda i: (0,)),
    ],
    out_specs=[
        pl.BlockSpec((NBLK, H), lambda i: (i, 0)),
        pl.BlockSpec((NBLK, H), lambda i: (i, 0)),
    ],
    out_shape=[
        jax.ShapeDtypeStruct((N, H), jnp.float32),
        jax.ShapeDtypeStruct((N, H), jnp.float32),
    ],
)


def _mlp_body(x_ref, w_ref, b_ref, m_ref):
    y = jnp.maximum(x_ref[...], 0.0)
    m_ref[...] = (jnp.dot(y, w_ref[...], preferred_element_type=jnp.float32)
                  + b_ref[...])


_mlp = pl.pallas_call(
    _mlp_body,
    grid=(E // NBLK,),
    in_specs=[
        pl.BlockSpec((NBLK, H), lambda i: (i, 0)),
        pl.BlockSpec((H, H), lambda i: (0, 0)),
        pl.BlockSpec((H,), lambda i: (0,)),
    ],
    out_specs=pl.BlockSpec((NBLK, H), lambda i: (i, 0)),
    out_shape=jax.ShapeDtypeStruct((E, H), jnp.float32),
)


def _fin_body(par_ref, w_ref, b_ref, o_ref):
    red = jnp.max(par_ref[...], axis=0)
    o_ref[...] = (jnp.dot(red, w_ref[...], preferred_element_type=jnp.float32)
                  + b_ref[...])


_fin = pl.pallas_call(
    _fin_body,
    out_shape=jax.ShapeDtypeStruct((G, 1), jnp.float32),
)


def kernel(pos, edge_index, batch, W1a, b1a, W1b, b1b, W2a, b2a, W2b, b2b,
           Wout, bout):
    src = edge_index[0]
    dst = edge_index[1]
    counts, countsc = _hist(dst)
    perm = _bin(src, dst, counts)
    srcp = perm[:E]
    dstp = perm[E:]
    a1, b1n = _nt1(pos, W1a, b1a)
    x1 = _gx(a1, b1n, srcp, dstp, countsc)
    m1 = _mlp(x1, W1b, b1b)
    h1 = _smax1(m1, dstp, countsc, batch)[:N]
    a2, b2n = _nt2(h1, pos, W2a, b2a)
    x2 = _gx(a2, b2n, srcp, dstp, countsc)
    m2 = _mlp(x2, W2b, b2b)
    partials = _smax2(m2, dstp, countsc, batch)
    return _fin(partials, Wout, bout)


# speedup vs baseline: 1.4897x; 1.4891x over previous
import functools

import jax
import jax.numpy as jnp
import numpy as np
from jax import lax
from jax.experimental import pallas as pl
from jax.experimental.pallas import tpu as pltpu
from jax.experimental.pallas import tpu_sc as plsc

N = 100000
E = 1600000
H = 32
G = 64
NC = 2
NS = 16
NW = NC * NS
EPT = E // NW
VPT = EPT // 16
BMULT = 41
BSHIFT = 17
MAXROWS = 3197
ACCROWS = 3328
DUMROW = ACCROWS - 1
NPAD = 128
SCHUNK = 256
MCHUNK = 256
NBLK = 4000

_BSTARTS_NP = np.minimum(
    (np.arange(48, dtype=np.int64) * (1 << BSHIFT) + BMULT - 1) // BMULT, N
).astype(np.int32)


def _mesh():
    return plsc.VectorSubcoreMesh(core_axis_name="c", subcore_axis_name="s")


def _wid():
    return lax.axis_index("s") * NC + lax.axis_index("c")


def _take(x, i):
    return x.at[i].get(mode="promise_in_bounds")


def _iota():
    return lax.iota(jnp.int32, 16)


def _bucket(d):
    return (d * BMULT) >> BSHIFT


def _m8(x):
    return pl.multiple_of(x, 8)


def _vsum(v):
    iota = _iota()
    for d in (1, 2, 4, 8):
        v = v + _take(v, jnp.bitwise_xor(iota, d))
    return v


def _ssum(v):
    return _vsum(v)[0]


def _prefix_counts(cbuf, w):
    iota = _iota()
    zi = jnp.zeros((16,), jnp.int32)

    def pf(p, carry):
        s0, s1 = carry
        v = cbuf[pl.ds(_m8(p * 16), 16)]
        b = jnp.bitwise_and(p * 16 + iota, 31)
        s0 = s0 + jnp.where(b < w, v, zi)
        s1 = s1 + jnp.where(b == w, v, zi)
        return (s0, s1)

    s0, s1 = lax.fori_loop(0, NW * 32 // 16, pf, (zi, zi))
    return _ssum(s0), _ssum(s1)


def _hist_body(dst_hbm, counts_hbm, cc_hbm, dbuf, cnt, cntc):
    w = _wid()
    pltpu.sync_copy(dst_hbm.at[pl.ds(_m8(w * EPT), EPT)], dbuf)
    iota = _iota()
    z = jnp.zeros((16,), jnp.int32)
    ones = jnp.ones((16,), jnp.int32)
    for p in range(32):
        cnt[p, pl.ds(0, 16)] = z
    onehots = [jnp.where(iota == k, 1, 0) for k in range(16)]

    def chunk(c, _):
        def body(v, _):
            d = dbuf[pl.ds(_m8(c * 80 + v * 16), 16)]
            b = _bucket(d)
            for k in range(16):
                bk = b[k]
                cnt[bk, pl.ds(0, 16)] = cnt[bk, pl.ds(0, 16)] + onehots[k]
            return 0

        lax.fori_loop(0, 5, body, 0, unroll=True)
        return 0

    lax.fori_loop(0, VPT // 5, chunk, 0)
    pltpu.sync_copy(cnt, counts_hbm.at[pl.ds(_m8(w * 32), 32)])

    v0 = z
    v1 = z
    for b in range(32):
        s = _ssum(cnt[b, pl.ds(0, 16)])
        sel = jnp.where(iota == (b & 15), ones, z)
        if b < 16:
            v0 = v0 + s * sel
        else:
            v1 = v1 + s * sel
    cntc[pl.ds(0, 16)] = v0
    cntc[pl.ds(16, 16)] = v1
    pltpu.sync_copy(cntc, cc_hbm.at[pl.ds(_m8(w * 32), 32)])


_hist = pl.kernel(
    _hist_body,
    out_type=(
        jax.ShapeDtypeStruct((NW * 32, 16), jnp.int32),
        jax.ShapeDtypeStruct((NW * 32,), jnp.int32),
    ),
    mesh=_mesh(),
    compiler_params=pltpu.CompilerParams(use_tc_tiling_on_sc=False),
    scratch_types=[
        pltpu.VMEM((EPT,), jnp.int32),
        pltpu.VMEM((32, 16), jnp.int32),
        pltpu.VMEM((32,), jnp.int32),
    ],
)


def _bin_body(src_hbm, dst_hbm, counts_hbm, perm_hbm,
              sbuf, dbuf, cbuf, base, stg_pos, stg_dat, sem):
    w = _wid()
    pltpu.sync_copy(src_hbm.at[pl.ds(_m8(w * EPT), EPT)], sbuf)
    pltpu.sync_copy(dst_hbm.at[pl.ds(_m8(w * EPT), EPT)], dbuf)
    pltpu.sync_copy(counts_hbm, cbuf)
    iota = _iota()
    zi = jnp.zeros((16,), jnp.int32)

    def bb(b, gstart):
        def wacc(wp, carry):
            t, p_, m_ = carry
            v = cbuf[wp * 32 + b, pl.ds(0, 16)]
            fp = jnp.where(wp < w, 1, 0)
            fm = jnp.where(wp == w, 1, 0)
            return (t + v, p_ + v * fp, m_ + v * fm)

        totv, priorv, myv = lax.fori_loop(0, NW, wacc, (zi, zi, zi))
        inc = myv
        for dsh in (1, 2, 4, 8):
            g = _take(inc, jnp.maximum(iota - dsh, 0))
            inc = inc + jnp.where(iota >= dsh, g, zi)
        base[b, pl.ds(0, 16)] = gstart + _ssum(priorv) + (inc - myv)
        return gstart + _ssum(totv)

    lax.fori_loop(0, 32, bb, jnp.int32(0))
    onehots = [jnp.where(iota == k, 1, 0) for k in range(16)]

    def chunk(c, _):
        slot = lax.rem(c, 8)
        soff = _m8(slot * 160)

        @pl.when(c >= 8)
        def _wait_prev():
            pltpu.make_async_copy(
                stg_dat.at[pl.ds(soff, 160)],
                perm_hbm.at[stg_pos.at[slot]], sem).wait()

        def vbody(v, _):
            k = _m8(c * 80 + v * 16)
            s = sbuf[pl.ds(k, 16)]
            d = dbuf[pl.ds(k, 16)]
            b = _bucket(d)
            pos = jnp.zeros((16,), jnp.int32)
            for t in range(32):
                rowv = base[t, pl.ds(0, 16)]
                mi = jnp.where(b == t, 1, 0)
                pos = pos + rowv * mi
                base[t, pl.ds(0, 16)] = rowv + mi
            stg_pos[slot, pl.ds(v * 16, 16)] = pos
            stg_pos[slot, pl.ds(80 + v * 16, 16)] = pos + E
            stg_dat[pl.ds(_m8(soff + v * 16), 16)] = s
            stg_dat[pl.ds(_m8(soff + 80 + v * 16), 16)] = d
            return 0

        lax.fori_loop(0, 5, vbody, 0)
        pltpu.async_copy(
            stg_dat.at[pl.ds(soff, 160)],
            perm_hbm.at[stg_pos.at[slot]], sem)
        return 0

    lax.fori_loop(0, EPT // 80, chunk, 0)
    for sl in range(8):
        pltpu.make_async_copy(
            stg_dat.at[pl.ds(sl * 160, 160)],
            perm_hbm.at[stg_pos.at[sl]], sem).wait()


_bin = pl.kernel(
    _bin_body,
    out_type=jax.ShapeDtypeStruct((2 * E,), jnp.int32),
    mesh=_mesh(),
    compiler_params=pltpu.CompilerParams(use_tc_tiling_on_sc=False),
    scratch_types=[
        pltpu.VMEM((EPT,), jnp.int32),
        pltpu.VMEM((EPT,), jnp.int32),
        pltpu.VMEM((NW * 32, 16), jnp.int32),
        pltpu.VMEM((32, 16), jnp.int32),
        pltpu.VMEM((8, 160), jnp.int32),
        pltpu.VMEM((1280,), jnp.int32),
        pltpu.SemaphoreType.DMA,
    ],
)


def _gx_body(a_hbm, bn_hbm, srcp, dstp, counts_hbm, x_hbm,
             cbuf, sidx, didx, arows, brows, semi, semg):
    w = _wid()
    pltpu.sync_copy(counts_hbm, cbuf)
    start, nedges = _prefix_counts(cbuf, w)
    cstart = _m8(jnp.bitwise_and(start, -8))
    cend = _m8(jnp.bitwise_and(start + nedges + 7, -8))
    nch = (cend - cstart + SCHUNK - 1) // SCHUNK

    def offof(c):
        return _m8(jnp.maximum(0, jnp.minimum(cstart + c * SCHUNK,
                                              cend - SCHUNK)))

    def issue_idx(c):
        slot = lax.rem(c, 2)
        soff = _m8(slot * SCHUNK)
        off = offof(c)
        pltpu.async_copy(srcp.at[pl.ds(off, SCHUNK)],
                         sidx.at[pl.ds(soff, SCHUNK)], semi)
        pltpu.async_copy(dstp.at[pl.ds(off, SCHUNK)],
                         didx.at[pl.ds(soff, SCHUNK)], semi)

    @pl.when(nch > 0)
    def _prologue():
        issue_idx(0)

    def chunk(c, _):
        slot = lax.rem(c, 2)
        soff = _m8(slot * SCHUNK)
        pltpu.make_async_copy(
            srcp.at[pl.ds(0, SCHUNK)],
            sidx.at[pl.ds(0, SCHUNK)], semi).wait()
        pltpu.make_async_copy(
            srcp.at[pl.ds(0, SCHUNK)],
            didx.at[pl.ds(0, SCHUNK)], semi).wait()
        for hblk in range(SCHUNK // 128):
            pltpu.async_copy(
                a_hbm.at[sidx.at[pl.ds(_m8(soff + hblk * 128), 128)]],
                arows.at[pl.ds(_m8(soff + hblk * 128), 128)], semg)
            pltpu.async_copy(
                bn_hbm.at[didx.at[pl.ds(_m8(soff + hblk * 128), 128)]],
                brows.at[pl.ds(_m8(soff + hblk * 128), 128)], semg)

        @pl.when(c + 1 < nch)
        def _next():
            issue_idx(c + 1)

        for _ in range(2 * (SCHUNK // 128)):
            pltpu.make_async_copy(
                a_hbm.at[pl.ds(0, 128)],
                arows.at[pl.ds(0, 128)], semg).wait()

        @plsc.parallel_loop(0, SCHUNK, unroll=4)
        def _combine(r):
            arows[soff + r, pl.ds(0, 16)] = (
                arows[soff + r, pl.ds(0, 16)] + brows[soff + r, pl.ds(0, 16)])
            arows[soff + r, pl.ds(16, 16)] = (
                arows[soff + r, pl.ds(16, 16)] + brows[soff + r, pl.ds(16, 16)])

        pltpu.sync_copy(arows.at[pl.ds(soff, SCHUNK)],
                        x_hbm.at[pl.ds(offof(c), SCHUNK)])
        return 0

    lax.fori_loop(0, nch, chunk, 0)


_gx = pl.kernel(
    _gx_body,
    out_type=jax.ShapeDtypeStruct((E, H), jnp.float32),
    mesh=_mesh(),
    compiler_params=pltpu.CompilerParams(use_tc_tiling_on_sc=False),
    scratch_types=[
        pltpu.VMEM((NW * 32,), jnp.int32),
        pltpu.VMEM((2 * SCHUNK,), jnp.int32),
        pltpu.VMEM((2 * SCHUNK,), jnp.int32),
        pltpu.VMEM((2 * SCHUNK, H), jnp.float32),
        pltpu.VMEM((2 * SCHUNK, H), jnp.float32),
        pltpu.SemaphoreType.DMA,
        pltpu.SemaphoreType.DMA,
    ],
)


def _smax_body(final, m_hbm, dstp, counts_hbm, batch, out_hbm,
               cbuf, acc, mrows, didx, widx, gacc, bbuf, semi, semm):
    w = _wid()
    iota = _iota()
    pltpu.sync_copy(counts_hbm, cbuf)

    def _bst_at(k):
        return jnp.minimum((k * (1 << BSHIFT) + BMULT - 1) // BMULT, N)

    nstart = _bst_at(w)
    nend = _bst_at(w + 1)
    nrows = nend - nstart
    z16 = jnp.zeros((16,), jnp.float32)

    @plsc.parallel_loop(0, ACCROWS, unroll=8)
    def _zero(r):
        acc[r, pl.ds(0, 16)] = z16
        acc[r, pl.ds(16, 16)] = z16

    start, nedges = _prefix_counts(cbuf, w)
    cstart = _m8(jnp.bitwise_and(start, -8))
    cend = _m8(jnp.bitwise_and(start + nedges + 7, -8))
    nch = (cend - cstart + MCHUNK - 1) // MCHUNK

    def offof(c):
        return _m8(jnp.maximum(0, jnp.minimum(cstart + c * MCHUNK,
                                              cend - MCHUNK)))

    def issue(c):
        slot = lax.rem(c, 2)
        soff = _m8(slot * MCHUNK)
        off = offof(c)
        pltpu.async_copy(dstp.at[pl.ds(off, MCHUNK)],
                         didx.at[pl.ds(soff, MCHUNK)], semi)
        pltpu.async_copy(m_hbm.at[pl.ds(off, MCHUNK)],
                         mrows.at[pl.ds(soff, MCHUNK)], semm)

    @pl.when(nch > 0)
    def _prologue():
        issue(0)

    def chunk(c, _):
        slot = lax.rem(c, 2)
        soff = _m8(slot * MCHUNK)
        pltpu.make_async_copy(
            dstp.at[pl.ds(0, MCHUNK)],
            didx.at[pl.ds(0, MCHUNK)], semi).wait()
        pltpu.make_async_copy(
            m_hbm.at[pl.ds(0, MCHUNK)],
            mrows.at[pl.ds(0, MCHUNK)], semm).wait()

        @pl.when(c + 1 < nch)
        def _next():
            issue(c + 1)

        def eb(v, _):
            lv = didx[pl.ds(_m8(soff + v * 16), 16)] - nstart
            valid = (lv >= 0) & (lv < nrows)
            lv = jnp.where(valid, lv, DUMROW)
            for k in range(16):
                l = lv[k]
                e = soff + v * 16 + k
                acc[l, pl.ds(0, 16)] = jnp.maximum(
                    acc[l, pl.ds(0, 16)], mrows[e, pl.ds(0, 16)])
                acc[l, pl.ds(16, 16)] = jnp.maximum(
                    acc[l, pl.ds(16, 16)], mrows[e, pl.ds(16, 16)])
            return 0

        lax.fori_loop(0, MCHUNK // 16, eb, 0)
        return 0

    lax.fori_loop(0, nch, chunk, 0)

    if not final:
        nwch = (nrows + 127) // 128

        def wb(cc, _):
            roff = _m8(cc * 128)

            def ib(v, _):
                gi = nstart + roff + v * 16 + iota
                widx[pl.ds(v * 16, 16)] = jnp.where(
                    gi < nend, gi, N + jnp.bitwise_and(gi, NPAD - 1))
                return 0

            lax.fori_loop(0, 8, ib, 0, unroll=True)
            pltpu.async_copy(
                acc.at[pl.ds(roff, 128)], out_hbm.at[widx], semm).wait()
            return 0

        lax.fori_loop(0, nwch, wb, 0)
    else:
        @plsc.parallel_loop(0, G + 1)
        def _gz(r):
            gacc[r, pl.ds(0, 16)] = z16
            gacc[r, pl.ds(16, 16)] = z16

        bs_al = _m8(jnp.bitwise_and(nstart, -8))
        be_al = _m8(jnp.bitwise_and(nend + 7, -8))
        nbch = (be_al - bs_al + MCHUNK - 1) // MCHUNK

        def gb(cc, _):
            boff = _m8(jnp.maximum(
                0, jnp.minimum(bs_al + cc * MCHUNK, be_al - MCHUNK)))
            pltpu.sync_copy(batch.at[pl.ds(boff, MCHUNK)], bbuf)

            def nb(v, _):
                l = boff + v * 16 + iota - nstart
                valid = (l >= 0) & (l < nrows)
                gv = jnp.where(valid, bbuf[pl.ds(v * 16, 16)], G)
                lv = jnp.where(valid, l, DUMROW)
                for k in range(16):
                    g = gv[k]
                    lk = lv[k]
                    gacc[g, pl.ds(0, 16)] = jnp.maximum(
                        gacc[g, pl.ds(0, 16)], acc[lk, pl.ds(0, 16)])
                    gacc[g, pl.ds(16, 16)] = jnp.maximum(
                        gacc[g, pl.ds(16, 16)], acc[lk, pl.ds(16, 16)])
                return 0

            lax.fori_loop(0, MCHUNK // 16, nb, 0)
            return 0

        lax.fori_loop(0, nbch, gb, 0)
        pltpu.sync_copy(gacc.at[pl.ds(0, G)], out_hbm.at[w])


def _make_smax(final):
    out_type = (jax.ShapeDtypeStruct((NW, G, H), jnp.float32) if final
                else jax.ShapeDtypeStruct((N + NPAD, H), jnp.float32))
    return pl.kernel(
        functools.partial(_smax_body, final),
        out_type=out_type,
        mesh=_mesh(),
        compiler_params=pltpu.CompilerParams(use_tc_tiling_on_sc=False),
        scratch_types=[
            pltpu.VMEM((NW * 32,), jnp.int32),
            pltpu.VMEM((ACCROWS, H), jnp.float32),
            pltpu.VMEM((2 * MCHUNK, H), jnp.float32),
            pltpu.VMEM((2 * MCHUNK,), jnp.int32),
            pltpu.VMEM((128,), jnp.int32),
            pltpu.VMEM((G + 1, H), jnp.float32),
            pltpu.VMEM((MCHUNK,), jnp.int32),
            pltpu.SemaphoreType.DMA,
            pltpu.SemaphoreType.DMA,
        ],
    )


_smax1 = _make_smax(False)
_smax2 = _make_smax(True)


def _nt1_body(pos_ref, w_ref, b_ref, a_ref, bn_ref):
    p = pos_ref[...]
    wfull = w_ref[...]
    wh = wfull[0:3]
    wp = wfull[3:6]
    p0, p1, p2 = p[:, 0:1], p[:, 1:2], p[:, 2:3]
    bp = p0 * wp[0] + p1 * wp[1] + p2 * wp[2]
    ah = p0 * wh[0] + p1 * wh[1] + p2 * wh[2]
    a_ref[...] = ah + bp + b_ref[...]
    bn_ref[...] = -bp


_nt1 = pl.pallas_call(
    _nt1_body,
    grid=(N // NBLK,),
    in_specs=[
        pl.BlockSpec((NBLK, 3), lambda i: (i, 0)),
        pl.BlockSpec((6, H), lambda i: (0, 0)),
        pl.BlockSpec((H,), lambda i: (0,)),
    ],
    out_specs=[
        pl.BlockSpec((NBLK, H), lambda i: (i, 0)),
        pl.BlockSpec((NBLK, H), lambda i: (i, 0)),
    ],
    out_shape=[
        jax.ShapeDtypeStruct((N, H), jnp.float32),
        jax.ShapeDtypeStruct((N, H), jnp.float32),
    ],
)


def _nt2_body(h_ref, pos_ref, w_ref, b_ref, a_ref, bn_ref):
    h = h_ref[...]
    p = pos_ref[...]
    wfull = w_ref[...]
    wh = wfull[0:H]
    wp = wfull[H:H + 3]
    bp = p[:, 0:1] * wp[0] + p[:, 1:2] * wp[1] + p[:, 2:3] * wp[2]
    a_ref[...] = (jnp.dot(h, wh, preferred_element_type=jnp.float32)
                  + bp + b_ref[...])
    bn_ref[...] = -bp


_nt2 = pl.pallas_call(
    _nt2_body,
    grid=(N // NBLK,),
    in_specs=[
        pl.BlockSpec((NBLK, H), lambda i: (i, 0)),
        pl.BlockSpec((NBLK, 3), lambda i: (i, 0)),
        pl.BlockSpec((H + 3, H), lambda i: (0, 0)),
        pl.BlockSpec((H,), lambda i: (0,)),
    ],
    out_specs=[
        pl.BlockSpec((NBLK, H), lambda i: (i, 0)),
        pl.BlockSpec((NBLK, H), lambda i: (i, 0)),
    ],
    out_shape=[
        jax.ShapeDtypeStruct((N, H), jnp.float32),
        jax.ShapeDtypeStruct((N, H), jnp.float32),
    ],
)


def _mlp_body(x_ref, w_ref, b_ref, m_ref):
    y = jnp.maximum(x_ref[...], 0.0)
    m_ref[...] = (jnp.dot(y, w_ref[...], preferred_element_type=jnp.float32)
                  + b_ref[...])


_mlp = pl.pallas_call(
    _mlp_body,
    grid=(E // 4 // NBLK,),
    in_specs=[
        pl.BlockSpec((NBLK, 4 * H), lambda i: (i, 0)),
        pl.BlockSpec((4 * H, 4 * H), lambda i: (0, 0)),
        pl.BlockSpec((4 * H,), lambda i: (0,)),
    ],
    out_specs=pl.BlockSpec((NBLK, 4 * H), lambda i: (i, 0)),
    out_shape=jax.ShapeDtypeStruct((E // 4, 4 * H), jnp.float32),
)


def _mlp4(x, wb, bb):
    w4 = jnp.kron(jnp.eye(4, dtype=jnp.float32), wb)
    b4 = jnp.tile(bb, 4)
    return _mlp(x.reshape(E // 4, 4 * H), w4, b4).reshape(E, H)


def _fin_body(par_ref, w_ref, b_ref, o_ref):
    red = jnp.max(par_ref[...], axis=0)
    o_ref[...] = (jnp.dot(red, w_ref[...], preferred_element_type=jnp.float32)
                  + b_ref[...])


_fin = pl.pallas_call(
    _fin_body,
    out_shape=jax.ShapeDtypeStruct((G, 1), jnp.float32),
)


def kernel(pos, edge_index, batch, W1a, b1a, W1b, b1b, W2a, b2a, W2b, b2b,
           Wout, bout):
    src = edge_index[0]
    dst = edge_index[1]
    counts, countsc = _hist(dst)
    perm = _bin(src, dst, counts)
    srcp = perm[:E]
    dstp = perm[E:]
    a1, b1n = _nt1(pos, W1a, b1a)
    x1 = _gx(a1, b1n, srcp, dstp, countsc)
    m1 = _mlp4(x1, W1b, b1b)
    h1 = _smax1(m1, dstp, countsc, batch)[:N]
    a2, b2n = _nt2(h1, pos, W2a, b2a)
    x2 = _gx(a2, b2n, srcp, dstp, countsc)
    m2 = _mlp4(x2, W2b, b2b)
    partials = _smax2(m2, dstp, countsc, batch)
    return _fin(partials, Wout, bout)


# speedup vs baseline: 1.5305x; 1.0274x over previous
import functools

import jax
import jax.numpy as jnp
import numpy as np
from jax import lax
from jax.experimental import pallas as pl
from jax.experimental.pallas import tpu as pltpu
from jax.experimental.pallas import tpu_sc as plsc

N = 100000
E = 1600000
H = 32
G = 64
NC = 2
NS = 16
NW = NC * NS
EPT = E // NW
VPT = EPT // 16
BMULT = 41
BSHIFT = 17
MAXROWS = 3197
ACCROWS = 3328
DUMROW = ACCROWS - 1
NPAD = 128
SCHUNK = 512
MCHUNK = 256
NBLK = 4000

_BSTARTS_NP = np.minimum(
    (np.arange(48, dtype=np.int64) * (1 << BSHIFT) + BMULT - 1) // BMULT, N
).astype(np.int32)


def _mesh():
    return plsc.VectorSubcoreMesh(core_axis_name="c", subcore_axis_name="s")


def _wid():
    return lax.axis_index("s") * NC + lax.axis_index("c")


def _take(x, i):
    return x.at[i].get(mode="promise_in_bounds")


def _iota():
    return lax.iota(jnp.int32, 16)


def _bucket(d):
    return (d * BMULT) >> BSHIFT


def _m8(x):
    return pl.multiple_of(x, 8)


def _vsum(v):
    iota = _iota()
    for d in (1, 2, 4, 8):
        v = v + _take(v, jnp.bitwise_xor(iota, d))
    return v


def _ssum(v):
    return _vsum(v)[0]


def _prefix_counts(cbuf, w):
    iota = _iota()
    zi = jnp.zeros((16,), jnp.int32)

    def pf(p, carry):
        s0, s1 = carry
        v = cbuf[pl.ds(_m8(p * 16), 16)]
        b = jnp.bitwise_and(p * 16 + iota, 31)
        s0 = s0 + jnp.where(b < w, v, zi)
        s1 = s1 + jnp.where(b == w, v, zi)
        return (s0, s1)

    s0, s1 = lax.fori_loop(0, NW * 32 // 16, pf, (zi, zi))
    return _ssum(s0), _ssum(s1)


def _hist_body(dst_hbm, counts_hbm, cc_hbm, dbuf, cnt, cntc):
    w = _wid()
    pltpu.sync_copy(dst_hbm.at[pl.ds(_m8(w * EPT), EPT)], dbuf)
    iota = _iota()
    z = jnp.zeros((16,), jnp.int32)
    ones = jnp.ones((16,), jnp.int32)
    for p in range(32):
        cnt[p, pl.ds(0, 16)] = z
    onehots = [jnp.where(iota == k, 1, 0) for k in range(16)]

    def chunk(c, _):
        def body(v, _):
            d = dbuf[pl.ds(_m8(c * 80 + v * 16), 16)]
            b = _bucket(d)
            for k in range(16):
                bk = b[k]
                cnt[bk, pl.ds(0, 16)] = cnt[bk, pl.ds(0, 16)] + onehots[k]
            return 0

        lax.fori_loop(0, 5, body, 0, unroll=True)
        return 0

    lax.fori_loop(0, VPT // 5, chunk, 0)
    pltpu.sync_copy(cnt, counts_hbm.at[pl.ds(_m8(w * 32), 32)])

    v0 = z
    v1 = z
    for b in range(32):
        s = _ssum(cnt[b, pl.ds(0, 16)])
        sel = jnp.where(iota == (b & 15), ones, z)
        if b < 16:
            v0 = v0 + s * sel
        else:
            v1 = v1 + s * sel
    cntc[pl.ds(0, 16)] = v0
    cntc[pl.ds(16, 16)] = v1
    pltpu.sync_copy(cntc, cc_hbm.at[pl.ds(_m8(w * 32), 32)])


_hist = pl.kernel(
    _hist_body,
    out_type=(
        jax.ShapeDtypeStruct((NW * 32, 16), jnp.int32),
        jax.ShapeDtypeStruct((NW * 32,), jnp.int32),
    ),
    mesh=_mesh(),
    compiler_params=pltpu.CompilerParams(use_tc_tiling_on_sc=False),
    scratch_types=[
        pltpu.VMEM((EPT,), jnp.int32),
        pltpu.VMEM((32, 16), jnp.int32),
        pltpu.VMEM((32,), jnp.int32),
    ],
)


def _bin_body(src_hbm, dst_hbm, counts_hbm, perm_hbm,
              sbuf, dbuf, cbuf, base, stg_pos, stg_dat, sem):
    w = _wid()
    pltpu.sync_copy(src_hbm.at[pl.ds(_m8(w * EPT), EPT)], sbuf)
    pltpu.sync_copy(dst_hbm.at[pl.ds(_m8(w * EPT), EPT)], dbuf)
    pltpu.sync_copy(counts_hbm, cbuf)
    iota = _iota()
    zi = jnp.zeros((16,), jnp.int32)

    def bb(b, gstart):
        def wacc(wp, carry):
            t, p_, m_ = carry
            v = cbuf[wp * 32 + b, pl.ds(0, 16)]
            fp = jnp.where(wp < w, 1, 0)
            fm = jnp.where(wp == w, 1, 0)
            return (t + v, p_ + v * fp, m_ + v * fm)

        totv, priorv, myv = lax.fori_loop(0, NW, wacc, (zi, zi, zi))
        inc = myv
        for dsh in (1, 2, 4, 8):
            g = _take(inc, jnp.maximum(iota - dsh, 0))
            inc = inc + jnp.where(iota >= dsh, g, zi)
        base[b, pl.ds(0, 16)] = gstart + _ssum(priorv) + (inc - myv)
        return gstart + _ssum(totv)

    lax.fori_loop(0, 32, bb, jnp.int32(0))
    onehots = [jnp.where(iota == k, 1, 0) for k in range(16)]

    def chunk(c, _):
        slot = lax.rem(c, 8)
        soff = _m8(slot * 160)

        @pl.when(c >= 8)
        def _wait_prev():
            pltpu.make_async_copy(
                stg_dat.at[pl.ds(soff, 160)],
                perm_hbm.at[stg_pos.at[slot]], sem).wait()

        def vbody(v, _):
            k = _m8(c * 80 + v * 16)
            s = sbuf[pl.ds(k, 16)]
            d = dbuf[pl.ds(k, 16)]
            b = _bucket(d)
            pos = jnp.zeros((16,), jnp.int32)
            for t in range(32):
                rowv = base[t, pl.ds(0, 16)]
                mi = jnp.where(b == t, 1, 0)
                pos = pos + rowv * mi
                base[t, pl.ds(0, 16)] = rowv + mi
            stg_pos[slot, pl.ds(v * 16, 16)] = pos
            stg_pos[slot, pl.ds(80 + v * 16, 16)] = pos + E
            stg_dat[pl.ds(_m8(soff + v * 16), 16)] = s
            stg_dat[pl.ds(_m8(soff + 80 + v * 16), 16)] = d
            return 0

        lax.fori_loop(0, 5, vbody, 0)
        pltpu.async_copy(
            stg_dat.at[pl.ds(soff, 160)],
            perm_hbm.at[stg_pos.at[slot]], sem)
        return 0

    lax.fori_loop(0, EPT // 80, chunk, 0)
    for sl in range(8):
        pltpu.make_async_copy(
            stg_dat.at[pl.ds(sl * 160, 160)],
            perm_hbm.at[stg_pos.at[sl]], sem).wait()


_bin = pl.kernel(
    _bin_body,
    out_type=jax.ShapeDtypeStruct((2 * E,), jnp.int32),
    mesh=_mesh(),
    compiler_params=pltpu.CompilerParams(use_tc_tiling_on_sc=False),
    scratch_types=[
        pltpu.VMEM((EPT,), jnp.int32),
        pltpu.VMEM((EPT,), jnp.int32),
        pltpu.VMEM((NW * 32, 16), jnp.int32),
        pltpu.VMEM((32, 16), jnp.int32),
        pltpu.VMEM((8, 160), jnp.int32),
        pltpu.VMEM((1280,), jnp.int32),
        pltpu.SemaphoreType.DMA,
    ],
)


def _gx_body(a_hbm, bn_hbm, srcp, dstp, counts_hbm, x_hbm,
             cbuf, sidx, didx, arows, brows, semi, semg):
    w = _wid()
    pltpu.sync_copy(counts_hbm, cbuf)
    start, nedges = _prefix_counts(cbuf, w)
    cstart = _m8(jnp.bitwise_and(start, -8))
    cend = _m8(jnp.bitwise_and(start + nedges + 7, -8))
    nch = (cend - cstart + SCHUNK - 1) // SCHUNK

    def offof(c):
        return _m8(jnp.maximum(0, jnp.minimum(cstart + c * SCHUNK,
                                              cend - SCHUNK)))

    def issue_idx(c):
        slot = lax.rem(c, 2)
        soff = _m8(slot * SCHUNK)
        off = offof(c)
        pltpu.async_copy(srcp.at[pl.ds(off, SCHUNK)],
                         sidx.at[pl.ds(soff, SCHUNK)], semi)
        pltpu.async_copy(dstp.at[pl.ds(off, SCHUNK)],
                         didx.at[pl.ds(soff, SCHUNK)], semi)

    @pl.when(nch > 0)
    def _prologue():
        issue_idx(0)

    def chunk(c, _):
        slot = lax.rem(c, 2)
        soff = _m8(slot * SCHUNK)
        pltpu.make_async_copy(
            srcp.at[pl.ds(0, SCHUNK)],
            sidx.at[pl.ds(0, SCHUNK)], semi).wait()
        pltpu.make_async_copy(
            srcp.at[pl.ds(0, SCHUNK)],
            didx.at[pl.ds(0, SCHUNK)], semi).wait()
        for hblk in range(SCHUNK // 128):
            pltpu.async_copy(
                a_hbm.at[sidx.at[pl.ds(_m8(soff + hblk * 128), 128)]],
                arows.at[pl.ds(_m8(soff + hblk * 128), 128)], semg)
            pltpu.async_copy(
                bn_hbm.at[didx.at[pl.ds(_m8(soff + hblk * 128), 128)]],
                brows.at[pl.ds(_m8(soff + hblk * 128), 128)], semg)

        @pl.when(c + 1 < nch)
        def _next():
            issue_idx(c + 1)

        for _ in range(2 * (SCHUNK // 128)):
            pltpu.make_async_copy(
                a_hbm.at[pl.ds(0, 128)],
                arows.at[pl.ds(0, 128)], semg).wait()

        @plsc.parallel_loop(0, SCHUNK, unroll=4)
        def _combine(r):
            arows[soff + r, pl.ds(0, 16)] = (
                arows[soff + r, pl.ds(0, 16)] + brows[soff + r, pl.ds(0, 16)])
            arows[soff + r, pl.ds(16, 16)] = (
                arows[soff + r, pl.ds(16, 16)] + brows[soff + r, pl.ds(16, 16)])

        pltpu.sync_copy(arows.at[pl.ds(soff, SCHUNK)],
                        x_hbm.at[pl.ds(offof(c), SCHUNK)])
        return 0

    lax.fori_loop(0, nch, chunk, 0)


_gx = pl.kernel(
    _gx_body,
    out_type=jax.ShapeDtypeStruct((E, H), jnp.float32),
    mesh=_mesh(),
    compiler_params=pltpu.CompilerParams(use_tc_tiling_on_sc=False),
    scratch_types=[
        pltpu.VMEM((NW * 32,), jnp.int32),
        pltpu.VMEM((2 * SCHUNK,), jnp.int32),
        pltpu.VMEM((2 * SCHUNK,), jnp.int32),
        pltpu.VMEM((2 * SCHUNK, H), jnp.float32),
        pltpu.VMEM((2 * SCHUNK, H), jnp.float32),
        pltpu.SemaphoreType.DMA,
        pltpu.SemaphoreType.DMA,
    ],
)


def _smax_body(final, m_hbm, dstp, counts_hbm, batch, out_hbm,
               cbuf, acc, mrows, didx, widx, gacc, bbuf, semi, semm):
    w = _wid()
    iota = _iota()
    pltpu.sync_copy(counts_hbm, cbuf)

    def _bst_at(k):
        return jnp.minimum((k * (1 << BSHIFT) + BMULT - 1) // BMULT, N)

    nstart = _bst_at(w)
    nend = _bst_at(w + 1)
    nrows = nend - nstart
    z16 = jnp.zeros((16,), jnp.float32)

    @plsc.parallel_loop(0, ACCROWS, unroll=8)
    def _zero(r):
        acc[r, pl.ds(0, 16)] = z16
        acc[r, pl.ds(16, 16)] = z16

    start, nedges = _prefix_counts(cbuf, w)
    cstart = _m8(jnp.bitwise_and(start, -8))
    cend = _m8(jnp.bitwise_and(start + nedges + 7, -8))
    nch = (cend - cstart + MCHUNK - 1) // MCHUNK

    def offof(c):
        return _m8(jnp.maximum(0, jnp.minimum(cstart + c * MCHUNK,
                                              cend - MCHUNK)))

    def issue(c):
        slot = lax.rem(c, 2)
        soff = _m8(slot * MCHUNK)
        off = offof(c)
        pltpu.async_copy(dstp.at[pl.ds(off, MCHUNK)],
                         didx.at[pl.ds(soff, MCHUNK)], semi)
        pltpu.async_copy(m_hbm.at[pl.ds(off, MCHUNK)],
                         mrows.at[pl.ds(soff, MCHUNK)], semm)

    @pl.when(nch > 0)
    def _prologue():
        issue(0)

    def chunk(c, _):
        slot = lax.rem(c, 2)
        soff = _m8(slot * MCHUNK)
        pltpu.make_async_copy(
            dstp.at[pl.ds(0, MCHUNK)],
            didx.at[pl.ds(0, MCHUNK)], semi).wait()
        pltpu.make_async_copy(
            m_hbm.at[pl.ds(0, MCHUNK)],
            mrows.at[pl.ds(0, MCHUNK)], semm).wait()

        @pl.when(c + 1 < nch)
        def _next():
            issue(c + 1)

        def eb(v, _):
            lv = didx[pl.ds(_m8(soff + v * 16), 16)] - nstart
            valid = (lv >= 0) & (lv < nrows)
            lv = jnp.where(valid, lv, DUMROW)
            for k in range(16):
                l = lv[k]
                e = soff + v * 16 + k
                acc[l, pl.ds(0, 16)] = jnp.maximum(
                    acc[l, pl.ds(0, 16)], mrows[e, pl.ds(0, 16)])
                acc[l, pl.ds(16, 16)] = jnp.maximum(
                    acc[l, pl.ds(16, 16)], mrows[e, pl.ds(16, 16)])
            return 0

        lax.fori_loop(0, MCHUNK // 16, eb, 0)
        return 0

    lax.fori_loop(0, nch, chunk, 0)

    if not final:
        nwch = (nrows + 127) // 128

        def wb(cc, _):
            roff = _m8(cc * 128)

            def ib(v, _):
                gi = nstart + roff + v * 16 + iota
                widx[pl.ds(v * 16, 16)] = jnp.where(
                    gi < nend, gi, N + jnp.bitwise_and(gi, NPAD - 1))
                return 0

            lax.fori_loop(0, 8, ib, 0, unroll=True)
            pltpu.async_copy(
                acc.at[pl.ds(roff, 128)], out_hbm.at[widx], semm).wait()
            return 0

        lax.fori_loop(0, nwch, wb, 0)
    else:
        @plsc.parallel_loop(0, G + 1)
        def _gz(r):
            gacc[r, pl.ds(0, 16)] = z16
            gacc[r, pl.ds(16, 16)] = z16

        bs_al = _m8(jnp.bitwise_and(nstart, -8))
        be_al = _m8(jnp.bitwise_and(nend + 7, -8))
        nbch = (be_al - bs_al + MCHUNK - 1) // MCHUNK

        def gb(cc, _):
            boff = _m8(jnp.maximum(
                0, jnp.minimum(bs_al + cc * MCHUNK, be_al - MCHUNK)))
            pltpu.sync_copy(batch.at[pl.ds(boff, MCHUNK)], bbuf)

            def nb(v, _):
                l = boff + v * 16 + iota - nstart
                valid = (l >= 0) & (l < nrows)
                gv = jnp.where(valid, bbuf[pl.ds(v * 16, 16)], G)
                lv = jnp.where(valid, l, DUMROW)
                for k in range(16):
                    g = gv[k]
                    lk = lv[k]
                    gacc[g, pl.ds(0, 16)] = jnp.maximum(
                        gacc[g, pl.ds(0, 16)], acc[lk, pl.ds(0, 16)])
                    gacc[g, pl.ds(16, 16)] = jnp.maximum(
                        gacc[g, pl.ds(16, 16)], acc[lk, pl.ds(16, 16)])
                return 0

            lax.fori_loop(0, MCHUNK // 16, nb, 0)
            return 0

        lax.fori_loop(0, nbch, gb, 0)
        pltpu.sync_copy(gacc.at[pl.ds(0, G)], out_hbm.at[w])


def _make_smax(final):
    out_type = (jax.ShapeDtypeStruct((NW, G, H), jnp.float32) if final
                else jax.ShapeDtypeStruct((N + NPAD, H), jnp.float32))
    return pl.kernel(
        functools.partial(_smax_body, final),
        out_type=out_type,
        mesh=_mesh(),
        compiler_params=pltpu.CompilerParams(use_tc_tiling_on_sc=False),
        scratch_types=[
            pltpu.VMEM((NW * 32,), jnp.int32),
            pltpu.VMEM((ACCROWS, H), jnp.float32),
            pltpu.VMEM((2 * MCHUNK, H), jnp.float32),
            pltpu.VMEM((2 * MCHUNK,), jnp.int32),
            pltpu.VMEM((128,), jnp.int32),
            pltpu.VMEM((G + 1, H), jnp.float32),
            pltpu.VMEM((MCHUNK,), jnp.int32),
            pltpu.SemaphoreType.DMA,
            pltpu.SemaphoreType.DMA,
        ],
    )


_smax1 = _make_smax(False)
_smax2 = _make_smax(True)


def _nt1_body(pos_ref, w_ref, b_ref, a_ref, bn_ref):
    p = pos_ref[...]
    wfull = w_ref[...]
    wh = wfull[0:3]
    wp = wfull[3:6]
    p0, p1, p2 = p[:, 0:1], p[:, 1:2], p[:, 2:3]
    bp = p0 * wp[0] + p1 * wp[1] + p2 * wp[2]
    ah = p0 * wh[0] + p1 * wh[1] + p2 * wh[2]
    a_ref[...] = ah + bp + b_ref[...]
    bn_ref[...] = -bp


_nt1 = pl.pallas_call(
    _nt1_body,
    grid=(N // NBLK,),
    in_specs=[
        pl.BlockSpec((NBLK, 3), lambda i: (i, 0)),
        pl.BlockSpec((6, H), lambda i: (0, 0)),
        pl.BlockSpec((H,), lambda i: (0,)),
    ],
    out_specs=[
        pl.BlockSpec((NBLK, H), lambda i: (i, 0)),
        pl.BlockSpec((NBLK, H), lambda i: (i, 0)),
    ],
    out_shape=[
        jax.ShapeDtypeStruct((N, H), jnp.float32),
        jax.ShapeDtypeStruct((N, H), jnp.float32),
    ],
)


def _nt2_body(h_ref, pos_ref, w_ref, b_ref, a_ref, bn_ref):
    h = h_ref[...]
    p = pos_ref[...]
    wfull = w_ref[...]
    wh = wfull[0:H]
    wp = wfull[H:H + 3]
    bp = p[:, 0:1] * wp[0] + p[:, 1:2] * wp[1] + p[:, 2:3] * wp[2]
    a_ref[...] = (jnp.dot(h, wh, preferred_element_type=jnp.float32)
                  + bp + b_ref[...])
    bn_ref[...] = -bp


_nt2 = pl.pallas_call(
    _nt2_body,
    grid=(N // NBLK,),
    in_specs=[
        pl.BlockSpec((NBLK, H), lambda i: (i, 0)),
        pl.BlockSpec((NBLK, 3), lambda i: (i, 0)),
        pl.BlockSpec((H + 3, H), lambda i: (0, 0)),
        pl.BlockSpec((H,), lambda i: (0,)),
    ],
    out_specs=[
        pl.BlockSpec((NBLK, H), lambda i: (i, 0)),
        pl.BlockSpec((NBLK, H), lambda i: (i, 0)),
    ],
    out_shape=[
        jax.ShapeDtypeStruct((N, H), jnp.float32),
        jax.ShapeDtypeStruct((N, H), jnp.float32),
    ],
)


def _mlp_body(x_ref, w_ref, b_ref, m_ref):
    y = jnp.maximum(x_ref[...], 0.0)
    m_ref[...] = (jnp.dot(y, w_ref[...], preferred_element_type=jnp.float32)
                  + b_ref[...])


_mlp = pl.pallas_call(
    _mlp_body,
    grid=(E // 4 // NBLK,),
    in_specs=[
        pl.BlockSpec((NBLK, 4 * H), lambda i: (i, 0)),
        pl.BlockSpec((4 * H, 4 * H), lambda i: (0, 0)),
        pl.BlockSpec((4 * H,), lambda i: (0,)),
    ],
    out_specs=pl.BlockSpec((NBLK, 4 * H), lambda i: (i, 0)),
    out_shape=jax.ShapeDtypeStruct((E // 4, 4 * H), jnp.float32),
)


def _mlp4(x, wb, bb):
    w4 = jnp.kron(jnp.eye(4, dtype=jnp.float32), wb)
    b4 = jnp.tile(bb, 4)
    return _mlp(x.reshape(E // 4, 4 * H), w4, b4).reshape(E, H)


def _fin_body(par_ref, w_ref, b_ref, o_ref):
    red = jnp.max(par_ref[...], axis=0)
    o_ref[...] = (jnp.dot(red, w_ref[...], preferred_element_type=jnp.float32)
                  + b_ref[...])


_fin = pl.pallas_call(
    _fin_body,
    out_shape=jax.ShapeDtypeStruct((G, 1), jnp.float32),
)


def kernel(pos, edge_index, batch, W1a, b1a, W1b, b1b, W2a, b2a, W2b, b2b,
           Wout, bout):
    src = edge_index[0]
    dst = edge_index[1]
    counts, countsc = _hist(dst)
    perm = _bin(src, dst, counts)
    srcp = perm[:E]
    dstp = perm[E:]
    a1, b1n = _nt1(pos, W1a, b1a)
    x1 = _gx(a1, b1n, srcp, dstp, countsc)
    m1 = _mlp4(x1, W1b, b1b)
    h1 = _smax1(m1, dstp, countsc, batch)[:N]
    a2, b2n = _nt2(h1, pos, W2a, b2a)
    x2 = _gx(a2, b2n, srcp, dstp, countsc)
    m2 = _mlp4(x2, W2b, b2b)
    partials = _smax2(m2, dstp, countsc, batch)
    return _fin(partials, Wout, bout)
